# Initial kernel scaffold; baseline (speedup 1.0000x reference)
import jax
import jax.numpy as jnp
from jax.experimental import pallas as pl


def kernel(tf_queries, queries, keys, values, mask, weights, weights1_real, weights1_imag):
    raise NotImplementedError("write your pallas kernel here")


# speedup vs baseline: 4.6711x; 4.6711x over previous
import functools

import numpy as np
import jax
import jax.numpy as jnp
from jax import lax
from jax.experimental import pallas as pl
from jax.experimental.pallas import tpu as pltpu
from jax.experimental.pallas import tpu_sc as plsc

B, L, H, E = 2, 2048, 8, 64
HE = H * E
SCALE = 1.0 / np.sqrt(512.0)
N_TOP = min(int(5 * np.ceil(np.log(L))), L)
N_PAD = 48
F_REAL = L // 2 + 1
F_PAD = 1280
NF = 5
FB = F_PAD // NF
NEG = -1e30


def _dft_mats():
    t = np.arange(L)[:, None].astype(np.float64)
    f = np.arange(F_REAL)[None, :].astype(np.float64)
    ang = 2.0 * np.pi * t * f / L
    wc = np.zeros((L, F_PAD), np.float32)
    ws = np.zeros((L, F_PAD), np.float32)
    wc[:, :F_REAL] = np.cos(ang)
    ws[:, :F_REAL] = -np.sin(ang)
    alpha = np.full((F_REAL,), 2.0)
    alpha[0] = 1.0
    alpha[-1] = 1.0
    vc = np.zeros((F_PAD, L), np.float32)
    vs = np.zeros((F_PAD, L), np.float32)
    vc[:F_REAL, :] = alpha[:, None] * np.cos(ang.T) / L
    vs[:F_REAL, :] = -alpha[:, None] * np.sin(ang.T) / L
    vs[0, :] = 0.0
    vs[F_REAL - 1, :] = 0.0
    return wc, ws, vc, vs


_WC, _WS, _VC, _VS = _dft_mats()
_HI = jax.lax.Precision.HIGHEST


def _amp_body(q_ref, k_ref, tf_ref, wc_ref, ws_ref, vc_ref, vs_ref,
              amp_t_ref, amp_tf_ref, acc_ref):
    fb = pl.program_id(1)
    qb = q_ref[0]
    kb = k_ref[0]
    wc = wc_ref[...]
    ws = ws_ref[...]
    dnT = (((0,), (0,)), ((), ()))
    qr = lax.dot_general(qb, wc, dnT, precision=_HI)
    qi = lax.dot_general(qb, ws, dnT, precision=_HI)
    kr = lax.dot_general(kb, wc, dnT, precision=_HI)
    ki = lax.dot_general(kb, ws, dnT, precision=_HI)
    ar = qr * kr + qi * ki
    ai = qi * kr - qr * ki
    inc = (lax.dot_general(ar, vc_ref[...], (((1,), (0,)), ((), ())),
                           precision=_HI)
           + lax.dot_general(ai, vs_ref[...], (((1,), (0,)), ((), ())),
                             precision=_HI))

    @pl.when(fb == 0)
    def _init():
        acc_ref[...] = inc
        tfb = tf_ref[0]
        amp_tf_ref[0] = jnp.sum(jnp.reshape(tfb * tfb, (H, E, L)), axis=1)

    @pl.when(fb != 0)
    def _acc():
        acc_ref[...] += inc

    @pl.when(fb == NF - 1)
    def _fin():
        c = acc_ref[...]
        amp_t_ref[0] = jnp.sum(jnp.reshape(c * c, (H, E, L)), axis=1)


def _amplitudes(xq, xk, xtf_t):
    grid = (B, NF)
    return pl.pallas_call(
        _amp_body,
        grid=grid,
        in_specs=[
            pl.BlockSpec((1, L, HE), lambda b, f: (b, 0, 0)),
            pl.BlockSpec((1, L, HE), lambda b, f: (b, 0, 0)),
            pl.BlockSpec((1, HE, L), lambda b, f: (b, 0, 0)),
            pl.BlockSpec((L, FB), lambda b, f: (0, f)),
            pl.BlockSpec((L, FB), lambda b, f: (0, f)),
            pl.BlockSpec((FB, L), lambda b, f: (f, 0)),
            pl.BlockSpec((FB, L), lambda b, f: (f, 0)),
        ],
        out_specs=[
            pl.BlockSpec((1, H, L), lambda b, f: (b, 0, 0)),
            pl.BlockSpec((1, H, L), lambda b, f: (b, 0, 0)),
        ],
        out_shape=[
            jax.ShapeDtypeStruct((B, H, L), jnp.float32),
            jax.ShapeDtypeStruct((B, H, L), jnp.float32),
        ],
        scratch_shapes=[pltpu.VMEM((HE, L), jnp.float32)],
    )(xq, xk, xtf_t, jnp.asarray(_WC), jnp.asarray(_WS),
      jnp.asarray(_VC), jnp.asarray(_VS))


_NCHUNK = L // 16


def _topk_gather_body(amp_t_hbm, amp_tf_hbm, qv_hbm, tfv_hbm,
                      qvsel_hbm, tfvsel_hbm,
                      amp_v, cmax_v, idx_v, rows_v, sem):
    wid = lax.axis_index("s") * 2 + lax.axis_index("c")
    br = wid // 16
    bh = wid - br * 16
    b = bh // H
    h = bh - b * H

    @pl.when(br == 0)
    def _():
        pltpu.sync_copy(amp_t_hbm.at[bh], amp_v)

    @pl.when(br == 1)
    def _():
        pltpu.sync_copy(amp_tf_hbm.at[bh], amp_v)

    lanes = lax.iota(jnp.int32, 16)
    mask0 = lanes == 0

    def _shuf(x, s):
        return x.at[lanes ^ s].get(mode="promise_in_bounds")

    def _vmax(x):
        for s in (8, 4, 2, 1):
            x = jnp.maximum(x, _shuf(x, s))
        return x

    def _vmin(x):
        for s in (8, 4, 2, 1):
            x = jnp.minimum(x, _shuf(x, s))
        return x

    neg_v = jnp.full((16,), NEG, jnp.float32)

    def _bld(g2, carry):
        vec = neg_v
        for j in range(16):
            s = _vmax(amp_v[pl.ds(pl.multiple_of((g2 * 16 + j) * 16, 16),
                                  16)])
            vec = jnp.where(lanes == j, s, vec)
        cmax_v[pl.ds(pl.multiple_of(g2 * 16, 16), 16)] = vec
        return carry

    lax.fori_loop(0, _NCHUNK // 16, _bld, 0)

    base = b * (L * H) + h
    base_v = jnp.full((16,), base, jnp.int32)
    for s0 in range(0, N_PAD, 16):
        idx_v[pl.ds(s0, 16)] = base_v

    def _sel(j, carry):
        m = cmax_v[pl.ds(0, 16)]
        for i in range(1, 8):
            m = jnp.maximum(m, cmax_v[pl.ds(i * 16, 16)])
        gmax_v = _vmax(m)
        big = jnp.full((16,), 1 << 30, jnp.int32)
        cand = big
        for i in range(8):
            cm = cmax_v[pl.ds(i * 16, 16)]
            cand = jnp.minimum(cand, jnp.where(cm >= gmax_v,
                                               i * 16 + lanes, big))
        chunk_id = _vmin(cand)[0]
        coff = pl.multiple_of(chunk_id * 16, 16)
        ch = amp_v[pl.ds(coff, 16)]
        lane = _vmin(jnp.where(ch >= gmax_v, lanes, big))[0]
        flat = chunk_id * 16 + lane
        gidx_v = jnp.full((16,), base + flat * H, jnp.int32)
        s0 = pl.multiple_of((j // 16) * 16, 16)
        islice = idx_v[pl.ds(s0, 16)]
        idx_v[pl.ds(s0, 16)] = jnp.where(lanes == j % 16, gidx_v, islice)
        newch = jnp.where(lanes == lane, neg_v, ch)
        amp_v[pl.ds(coff, 16)] = newch
        g2off = pl.multiple_of((chunk_id // 16) * 16, 16)
        cslice = cmax_v[pl.ds(g2off, 16)]
        cmax_v[pl.ds(g2off, 16)] = jnp.where(lanes == chunk_id % 16,
                                             _vmax(newch), cslice)
        return carry

    lax.fori_loop(0, N_TOP, _sel, 0)

    @pl.when(br == 0)
    def _():
        pltpu.async_copy(qv_hbm.at[idx_v], rows_v, sem).wait()
        pltpu.sync_copy(rows_v, qvsel_hbm.at[bh])

    @pl.when(br == 1)
    def _():
        pltpu.async_copy(tfv_hbm.at[idx_v], rows_v, sem).wait()
        pltpu.sync_copy(rows_v, tfvsel_hbm.at[bh])


def _topk_gather(amp_t, amp_tf, qv, tfv):
    mesh = plsc.VectorSubcoreMesh(core_axis_name="c", subcore_axis_name="s")
    sds = jax.ShapeDtypeStruct((16, N_PAD, 2 * E), jnp.float32)
    fn = pl.kernel(
        _topk_gather_body,
        mesh=mesh,
        out_type=[sds, sds],
        scratch_types=[
            pltpu.VMEM((L,), jnp.float32),
            pltpu.VMEM((_NCHUNK,), jnp.float32),
            pltpu.VMEM((N_PAD,), jnp.int32),
            pltpu.VMEM((N_PAD, 2 * E), jnp.float32),
            pltpu.SemaphoreType.DMA,
        ],
    )
    return fn(amp_t, amp_tf, qv, tfv)


def _attn_body(qv_ref, tfv_ref, k_ref, tf_ref, w_ref, out_ref):
    kb = k_ref[0]
    tfb = tf_ref[0]
    dnS = (((1,), (1,)), ((), ()))
    dnO = (((0,), (0,)), ((), ()))
    rows = lax.broadcasted_iota(jnp.int32, (N_PAD, L), 0)
    valid = rows < N_TOP

    def branch(sel, vsel, kmat):
        s = lax.dot_general(sel, kmat, dnS, precision=_HI) * SCALE
        s = jnp.where(valid, s, NEG)
        s = s - jnp.max(s, axis=0, keepdims=True)
        p = jnp.exp(s)
        a = p / jnp.sum(p, axis=0, keepdims=True)
        return lax.dot_general(vsel, a, dnO, precision=_HI)

    qv = qv_ref[0]
    tfv = tfv_ref[0]
    o_t = branch(qv[:, :E], qv[:, E:], kb)
    o_tf = branch(tfv[:, :E], tfv[:, E:], tfb)
    w0 = w_ref[0, 0, :]
    w1 = w_ref[0, 1, :]
    nw0 = 1.0 / (1.0 + jnp.exp(w1 - w0))
    out_ref[0, 0] = o_t * nw0[None, :] + o_tf * (1.0 - nw0)[None, :]


def _attention(qv_sel, tfv_sel, k_t, tf_t, w_t):
    grid = (B, H)
    return pl.pallas_call(
        _attn_body,
        grid=grid,
        in_specs=[
            pl.BlockSpec((1, N_PAD, 2 * E), lambda b, h: (b * H + h, 0, 0)),
            pl.BlockSpec((1, N_PAD, 2 * E), lambda b, h: (b * H + h, 0, 0)),
            pl.BlockSpec((1, L, E), lambda b, h: (b * H + h, 0, 0)),
            pl.BlockSpec((1, L, E), lambda b, h: (b * H + h, 0, 0)),
            pl.BlockSpec((1, 2, L), lambda b, h: (h, 0, 0)),
        ],
        out_specs=pl.BlockSpec((1, 1, E, L), lambda b, h: (b, h, 0, 0)),
        out_shape=jax.ShapeDtypeStruct((B, H, E, L), jnp.float32),
    )(qv_sel, tfv_sel, k_t, tf_t, w_t)


def kernel(tf_queries, queries, keys, values, mask, weights,
           weights1_real, weights1_imag):
    del mask, weights1_real, weights1_imag
    xq = queries.reshape(B, L, HE)
    xk = keys.reshape(B, L, HE)
    xtf_t = jnp.transpose(tf_queries, (0, 2, 3, 1)).reshape(B, HE, L)
    amp_t, amp_tf = _amplitudes(xq, xk, xtf_t)
    v_r = values.reshape(B * L * H, E)
    qv = jnp.concatenate([queries.reshape(B * L * H, E), v_r], axis=1)
    tfv = jnp.concatenate([tf_queries.reshape(B * L * H, E), v_r], axis=1)
    qv_sel, tfv_sel = _topk_gather(
        amp_t.reshape(B * H, L), amp_tf.reshape(B * H, L), qv, tfv)
    w_t = jnp.transpose(weights, (0, 2, 1))
    k_t = jnp.transpose(keys, (0, 2, 1, 3)).reshape(B * H, L, E)
    tf_t = jnp.transpose(tf_queries, (0, 2, 1, 3)).reshape(B * H, L, E)
    return _attention(qv_sel, tfv_sel, k_t, tf_t, w_t)


# speedup vs baseline: 4.9976x; 1.0699x over previous
import functools

import numpy as np
import jax
import jax.numpy as jnp
from jax import lax
from jax.experimental import pallas as pl
from jax.experimental.pallas import tpu as pltpu
from jax.experimental.pallas import tpu_sc as plsc

B, L, H, E = 2, 2048, 8, 64
HE = H * E
SCALE = 1.0 / np.sqrt(512.0)
N_TOP = min(int(5 * np.ceil(np.log(L))), L)
N_PAD = 48
F_REAL = L // 2 + 1
F_PAD = 1280
NF = 5
FB = F_PAD // NF
NEG = -1e30


def _dft_mats():
    t = np.arange(L)[:, None].astype(np.float64)
    f = np.arange(F_REAL)[None, :].astype(np.float64)
    ang = 2.0 * np.pi * t * f / L
    wc = np.zeros((L, F_PAD), np.float32)
    ws = np.zeros((L, F_PAD), np.float32)
    wc[:, :F_REAL] = np.cos(ang)
    ws[:, :F_REAL] = -np.sin(ang)
    alpha = np.full((F_REAL,), 2.0)
    alpha[0] = 1.0
    alpha[-1] = 1.0
    vc = np.zeros((F_PAD, L), np.float32)
    vs = np.zeros((F_PAD, L), np.float32)
    vc[:F_REAL, :] = alpha[:, None] * np.cos(ang.T) / L
    vs[:F_REAL, :] = -alpha[:, None] * np.sin(ang.T) / L
    vs[0, :] = 0.0
    vs[F_REAL - 1, :] = 0.0
    return wc, ws, vc, vs


_WC, _WS, _VC, _VS = _dft_mats()
_SEL = np.repeat(np.eye(H, dtype=np.float32), E, axis=1)
_HI = jax.lax.Precision.HIGHEST
_HC = jax.lax.Precision.DEFAULT


def _amp_body(q_ref, k_ref, tf_ref, wc_ref, ws_ref, vc_ref, vs_ref, sel_ref,
              amp_t_ref, amp_tf_ref, acc_ref):
    fb = pl.program_id(1)
    qb = q_ref[0]
    kb = k_ref[0]
    wc = wc_ref[...]
    ws = ws_ref[...]
    dnT = (((0,), (0,)), ((), ()))
    qr = lax.dot_general(qb, wc, dnT, precision=_HI)
    qi = lax.dot_general(qb, ws, dnT, precision=_HI)
    kr = lax.dot_general(kb, wc, dnT, precision=_HI)
    ki = lax.dot_general(kb, ws, dnT, precision=_HI)
    ar = qr * kr + qi * ki
    ai = qi * kr - qr * ki
    inc = (lax.dot_general(ar, vc_ref[...], (((1,), (0,)), ((), ())),
                           precision=_HI)
           + lax.dot_general(ai, vs_ref[...], (((1,), (0,)), ((), ())),
                             precision=_HI))

    @pl.when(fb == 0)
    def _init():
        acc_ref[...] = inc
        tfb = tf_ref[0]
        amp_tf_ref[0] = lax.dot_general(
            sel_ref[...], tfb * tfb, (((1,), (1,)), ((), ())),
            precision=_HI)

    @pl.when(fb != 0)
    def _acc():
        acc_ref[...] += inc

    @pl.when(fb == NF - 1)
    def _fin():
        c = acc_ref[...]
        amp_t_ref[0] = jnp.sum(jnp.reshape(c * c, (H, E, L)), axis=1)


def _amplitudes(xq, xk, xtf):
    grid = (B, NF)
    return pl.pallas_call(
        _amp_body,
        grid=grid,
        in_specs=[
            pl.BlockSpec((1, L, HE), lambda b, f: (b, 0, 0)),
            pl.BlockSpec((1, L, HE), lambda b, f: (b, 0, 0)),
            pl.BlockSpec((1, L, HE), lambda b, f: (b, 0, 0)),
            pl.BlockSpec((L, FB), lambda b, f: (0, f)),
            pl.BlockSpec((L, FB), lambda b, f: (0, f)),
            pl.BlockSpec((FB, L), lambda b, f: (f, 0)),
            pl.BlockSpec((FB, L), lambda b, f: (f, 0)),
            pl.BlockSpec((H, HE), lambda b, f: (0, 0)),
        ],
        out_specs=[
            pl.BlockSpec((1, H, L), lambda b, f: (b, 0, 0)),
            pl.BlockSpec((1, H, L), lambda b, f: (b, 0, 0)),
        ],
        out_shape=[
            jax.ShapeDtypeStruct((B, H, L), jnp.float32),
            jax.ShapeDtypeStruct((B, H, L), jnp.float32),
        ],
        scratch_shapes=[pltpu.VMEM((HE, L), jnp.float32)],
    )(xq, xk, xtf, jnp.asarray(_WC), jnp.asarray(_WS),
      jnp.asarray(_VC), jnp.asarray(_VS), jnp.asarray(_SEL))


_NCHUNK = L // 16


def _topk_gather_body(amp_t_hbm, amp_tf_hbm, qv_hbm, tfv_hbm,
                      qvsel_hbm, tfvsel_hbm,
                      amp_v, cmax_v, idx_v, rows_v, sem):
    wid = lax.axis_index("s") * 2 + lax.axis_index("c")
    br = wid // 16
    bh = wid - br * 16
    b = bh // H
    h = bh - b * H

    @pl.when(br == 0)
    def _():
        pltpu.sync_copy(amp_t_hbm.at[bh], amp_v)

    @pl.when(br == 1)
    def _():
        pltpu.sync_copy(amp_tf_hbm.at[bh], amp_v)

    lanes = lax.iota(jnp.int32, 16)
    mask0 = lanes == 0

    def _shuf(x, s):
        return x.at[lanes ^ s].get(mode="promise_in_bounds")

    def _vmax(x):
        for s in (8, 4, 2, 1):
            x = jnp.maximum(x, _shuf(x, s))
        return x

    def _vmin(x):
        for s in (8, 4, 2, 1):
            x = jnp.minimum(x, _shuf(x, s))
        return x

    neg_v = jnp.full((16,), NEG, jnp.float32)

    def _bld(g2, carry):
        vec = neg_v
        for j in range(16):
            s = _vmax(amp_v[pl.ds(pl.multiple_of((g2 * 16 + j) * 16, 16),
                                  16)])
            vec = jnp.where(lanes == j, s, vec)
        cmax_v[pl.ds(pl.multiple_of(g2 * 16, 16), 16)] = vec
        return carry

    lax.fori_loop(0, _NCHUNK // 16, _bld, 0)

    base = b * (L * H) + h
    base_v = jnp.full((16,), base, jnp.int32)
    for s0 in range(0, N_PAD, 16):
        idx_v[pl.ds(s0, 16)] = base_v

    def _sel(j, carry):
        m = cmax_v[pl.ds(0, 16)]
        for i in range(1, 8):
            m = jnp.maximum(m, cmax_v[pl.ds(i * 16, 16)])
        gmax_v = _vmax(m)
        big = jnp.full((16,), 1 << 30, jnp.int32)
        cand = big
        for i in range(8):
            cm = cmax_v[pl.ds(i * 16, 16)]
            cand = jnp.minimum(cand, jnp.where(cm >= gmax_v,
                                               i * 16 + lanes, big))
        chunk_id = _vmin(cand)[0]
        coff = pl.multiple_of(chunk_id * 16, 16)
        ch = amp_v[pl.ds(coff, 16)]
        lane = _vmin(jnp.where(ch >= gmax_v, lanes, big))[0]
        flat = chunk_id * 16 + lane
        gidx_v = jnp.full((16,), base + flat * H, jnp.int32)
        s0 = pl.multiple_of((j // 16) * 16, 16)
        islice = idx_v[pl.ds(s0, 16)]
        idx_v[pl.ds(s0, 16)] = jnp.where(lanes == j % 16, gidx_v, islice)
        newch = jnp.where(lanes == lane, neg_v, ch)
        amp_v[pl.ds(coff, 16)] = newch
        g2off = pl.multiple_of((chunk_id // 16) * 16, 16)
        cslice = cmax_v[pl.ds(g2off, 16)]
        cmax_v[pl.ds(g2off, 16)] = jnp.where(lanes == chunk_id % 16,
                                             _vmax(newch), cslice)
        return carry

    lax.fori_loop(0, N_TOP, _sel, 0)

    @pl.when(br == 0)
    def _():
        pltpu.async_copy(qv_hbm.at[idx_v], rows_v, sem).wait()
        pltpu.sync_copy(rows_v, qvsel_hbm.at[bh])

    @pl.when(br == 1)
    def _():
        pltpu.async_copy(tfv_hbm.at[idx_v], rows_v, sem).wait()
        pltpu.sync_copy(rows_v, tfvsel_hbm.at[bh])


def _topk_gather(amp_t, amp_tf, qv, tfv):
    mesh = plsc.VectorSubcoreMesh(core_axis_name="c", subcore_axis_name="s")
    sds = jax.ShapeDtypeStruct((16, N_PAD, 2 * E), jnp.float32)
    fn = pl.kernel(
        _topk_gather_body,
        mesh=mesh,
        out_type=[sds, sds],
        scratch_types=[
            pltpu.VMEM((L,), jnp.float32),
            pltpu.VMEM((_NCHUNK,), jnp.float32),
            pltpu.VMEM((N_PAD,), jnp.int32),
            pltpu.VMEM((N_PAD, 2 * E), jnp.float32),
            pltpu.SemaphoreType.DMA,
        ],
    )
    return fn(amp_t, amp_tf, qv, tfv)


def _attn_body(qv_ref, tfv_ref, k_ref, tf_ref, w_ref, out_ref):
    dnS = (((1,), (1,)), ((), ()))
    dnO = (((0,), (0,)), ((), ()))
    rows = lax.broadcasted_iota(jnp.int32, (N_PAD, L), 0)
    valid = rows < N_TOP

    def branch(sel, vsel, kmat):
        s = lax.dot_general(sel, kmat, dnS, precision=_HC) * SCALE
        s = jnp.where(valid, s, NEG)
        s = s - jnp.max(s, axis=0, keepdims=True)
        p = jnp.exp(s)
        a = p / jnp.sum(p, axis=0, keepdims=True)
        return lax.dot_general(vsel, a, dnO, precision=_HC)

    for s_ in range(2):
        kb = k_ref[0][:, s_ * E:(s_ + 1) * E]
        tfb = tf_ref[0][:, s_ * E:(s_ + 1) * E]
        qv = qv_ref[s_]
        tfv = tfv_ref[s_]
        o_t = branch(qv[:, :E], qv[:, E:], kb)
        o_tf = branch(tfv[:, :E], tfv[:, E:], tfb)
        w0 = w_ref[s_, 0, :]
        w1 = w_ref[s_, 1, :]
        nw0 = 1.0 / (1.0 + jnp.exp(w1 - w0))
        out_ref[0, s_] = o_t * nw0[None, :] + o_tf * (1.0 - nw0)[None, :]


def _attention(qv_sel, tfv_sel, xk, xtf, w_t):
    grid = (B, H // 2)
    return pl.pallas_call(
        _attn_body,
        grid=grid,
        in_specs=[
            pl.BlockSpec((2, N_PAD, 2 * E),
                         lambda b, j: (b * (H // 2) + j, 0, 0)),
            pl.BlockSpec((2, N_PAD, 2 * E),
                         lambda b, j: (b * (H // 2) + j, 0, 0)),
            pl.BlockSpec((1, L, 2 * E), lambda b, j: (b, 0, j)),
            pl.BlockSpec((1, L, 2 * E), lambda b, j: (b, 0, j)),
            pl.BlockSpec((2, 2, L), lambda b, j: (j, 0, 0)),
        ],
        out_specs=pl.BlockSpec((1, 2, E, L), lambda b, j: (b, j, 0, 0)),
        out_shape=jax.ShapeDtypeStruct((B, H, E, L), jnp.float32),
    )(qv_sel, tfv_sel, xk, xtf, w_t)


def kernel(tf_queries, queries, keys, values, mask, weights,
           weights1_real, weights1_imag):
    del mask, weights1_real, weights1_imag
    xq = queries.reshape(B, L, HE)
    xk = keys.reshape(B, L, HE)
    xtf = tf_queries.reshape(B, L, HE)
    amp_t, amp_tf = _amplitudes(xq, xk, xtf)
    v_r = values.reshape(B * L * H, E)
    qv = jnp.concatenate([queries.reshape(B * L * H, E), v_r], axis=1)
    tfv = jnp.concatenate([tf_queries.reshape(B * L * H, E), v_r], axis=1)
    qv_sel, tfv_sel = _topk_gather(
        amp_t.reshape(B * H, L), amp_tf.reshape(B * H, L), qv, tfv)
    w_t = jnp.transpose(weights, (0, 2, 1))
    return _attention(qv_sel, tfv_sel, xk, xtf, w_t)


# speedup vs baseline: 7.2301x; 1.4467x over previous
import functools

import numpy as np
import jax
import jax.numpy as jnp
from jax import lax
from jax.experimental import pallas as pl
from jax.experimental.pallas import tpu as pltpu
from jax.experimental.pallas import tpu_sc as plsc

B, L, H, E = 2, 2048, 8, 64
HE = H * E
SCALE = 1.0 / np.sqrt(512.0)
N_TOP = min(int(5 * np.ceil(np.log(L))), L)
N_PAD = 48
F_REAL = L // 2 + 1
F_PAD = 1280
NF = 5
FB = F_PAD // NF
NEG = -1e30


def _dft_mats():
    t = np.arange(L)[:, None].astype(np.float64)
    f = np.arange(F_REAL)[None, :].astype(np.float64)
    ang = 2.0 * np.pi * t * f / L
    wc = np.zeros((L, F_PAD), np.float32)
    ws = np.zeros((L, F_PAD), np.float32)
    wc[:, :F_REAL] = np.cos(ang)
    ws[:, :F_REAL] = -np.sin(ang)
    alpha = np.full((F_REAL,), 2.0)
    alpha[0] = 1.0
    alpha[-1] = 1.0
    vc = np.zeros((F_PAD, L), np.float32)
    vs = np.zeros((F_PAD, L), np.float32)
    vc[:F_REAL, :] = alpha[:, None] * np.cos(ang.T) / L
    vs[:F_REAL, :] = -alpha[:, None] * np.sin(ang.T) / L
    vs[0, :] = 0.0
    vs[F_REAL - 1, :] = 0.0
    return wc, ws, vc, vs


_WC, _WS, _VC, _VS = _dft_mats()
_WF = np.concatenate(
    [np.concatenate([_WC[:, i * FB:(i + 1) * FB],
                     _WS[:, i * FB:(i + 1) * FB]], axis=1)
     for i in range(NF)], axis=1)
_VI = np.concatenate(
    [np.concatenate([_VC[i * FB:(i + 1) * FB],
                     _VS[i * FB:(i + 1) * FB]], axis=0)
     for i in range(NF)], axis=0)


def _split_bf16(a):
    import ml_dtypes
    hi = a.astype(ml_dtypes.bfloat16)
    lo = (a - hi.astype(np.float32)).astype(ml_dtypes.bfloat16)
    return hi, lo


_WF_H, _WF_L = _split_bf16(_WF)
_VI_H, _VI_L = _split_bf16(_VI)
_SEL = np.repeat(np.eye(H, dtype=np.float32), E, axis=1)
_HI = jax.lax.Precision.HIGHEST
_HC = jax.lax.Precision.DEFAULT


def _dot3(ah, al, bh, bl, dn):
    f32 = jnp.float32
    return (lax.dot_general(ah, bh, dn, preferred_element_type=f32)
            + lax.dot_general(ah, bl, dn, preferred_element_type=f32)
            + lax.dot_general(al, bh, dn, preferred_element_type=f32))


def _hilo(x):
    hi = x.astype(jnp.bfloat16)
    lo = (x - hi.astype(jnp.float32)).astype(jnp.bfloat16)
    return hi, lo


def _amp_body(q_ref, k_ref, tf_ref, wfh_ref, wfl_ref, vih_ref, vil_ref,
              sel_ref, amp_t_ref, amp_tf_ref, acc_ref):
    fb = pl.program_id(1)
    dnT = (((0,), (0,)), ((), ()))
    dnN = (((1,), (0,)), ((), ()))
    qh, ql = _hilo(q_ref[0])
    kh, kl = _hilo(k_ref[0])
    qrs = _dot3(qh, ql, wfh_ref[...], wfl_ref[...], dnT)
    krs = _dot3(kh, kl, wfh_ref[...], wfl_ref[...], dnT)
    qr, qi = qrs[:, :FB], qrs[:, FB:]
    kr, ki = krs[:, :FB], krs[:, FB:]
    ar = qr * kr + qi * ki
    ai = qi * kr - qr * ki
    aa = jnp.concatenate([ar, ai], axis=1)
    aah, aal = _hilo(aa)
    inc = _dot3(aah, aal, vih_ref[...], vil_ref[...], dnN)

    @pl.when(fb == 0)
    def _init():
        acc_ref[...] = inc
        tfb = tf_ref[0]
        amp_tf_ref[0] = lax.dot_general(
            sel_ref[...], tfb * tfb, (((1,), (1,)), ((), ())),
            precision=_HI)

    @pl.when(fb != 0)
    def _acc():
        acc_ref[...] += inc

    @pl.when(fb == NF - 1)
    def _fin():
        c = acc_ref[...]
        amp_t_ref[0] = jnp.sum(jnp.reshape(c * c, (H, E, L)), axis=1)


def _amplitudes(xq, xk, xtf):
    grid = (B, NF)
    return pl.pallas_call(
        _amp_body,
        grid=grid,
        in_specs=[
            pl.BlockSpec((1, L, HE), lambda b, f: (b, 0, 0)),
            pl.BlockSpec((1, L, HE), lambda b, f: (b, 0, 0)),
            pl.BlockSpec((1, L, HE), lambda b, f: (b, 0, 0)),
            pl.BlockSpec((L, 2 * FB), lambda b, f: (0, f)),
            pl.BlockSpec((L, 2 * FB), lambda b, f: (0, f)),
            pl.BlockSpec((2 * FB, L), lambda b, f: (f, 0)),
            pl.BlockSpec((2 * FB, L), lambda b, f: (f, 0)),
            pl.BlockSpec((H, HE), lambda b, f: (0, 0)),
        ],
        out_specs=[
            pl.BlockSpec((1, H, L), lambda b, f: (b, 0, 0)),
            pl.BlockSpec((1, H, L), lambda b, f: (b, 0, 0)),
        ],
        out_shape=[
            jax.ShapeDtypeStruct((B, H, L), jnp.float32),
            jax.ShapeDtypeStruct((B, H, L), jnp.float32),
        ],
        scratch_shapes=[pltpu.VMEM((HE, L), jnp.float32)],
    )(xq, xk, xtf, jnp.asarray(_WF_H), jnp.asarray(_WF_L),
      jnp.asarray(_VI_H), jnp.asarray(_VI_L), jnp.asarray(_SEL))


_NCHUNK = L // 16


def _topk_gather_body(amp_t_hbm, amp_tf_hbm, qv_hbm, tfv_hbm,
                      qvsel_hbm, tfvsel_hbm,
                      amp_v, cmax_v, idx_v, rows_v, sem):
    wid = lax.axis_index("s") * 2 + lax.axis_index("c")
    br = wid // 16
    bh = wid - br * 16
    b = bh // H
    h = bh - b * H

    @pl.when(br == 0)
    def _():
        pltpu.sync_copy(amp_t_hbm.at[bh], amp_v)

    @pl.when(br == 1)
    def _():
        pltpu.sync_copy(amp_tf_hbm.at[bh], amp_v)

    lanes = lax.iota(jnp.int32, 16)
    mask0 = lanes == 0

    def _shuf(x, s):
        return x.at[lanes ^ s].get(mode="promise_in_bounds")

    def _vmax(x):
        for s in (8, 4, 2, 1):
            x = jnp.maximum(x, _shuf(x, s))
        return x

    def _vmin(x):
        for s in (8, 4, 2, 1):
            x = jnp.minimum(x, _shuf(x, s))
        return x

    neg_v = jnp.full((16,), NEG, jnp.float32)

    def _bld(g2, carry):
        vec = neg_v
        for j in range(16):
            s = _vmax(amp_v[pl.ds(pl.multiple_of((g2 * 16 + j) * 16, 16),
                                  16)])
            vec = jnp.where(lanes == j, s, vec)
        cmax_v[pl.ds(pl.multiple_of(g2 * 16, 16), 16)] = vec
        return carry

    lax.fori_loop(0, _NCHUNK // 16, _bld, 0)

    base = b * (L * H) + h
    base_v = jnp.full((16,), base, jnp.int32)
    for s0 in range(0, N_PAD, 16):
        idx_v[pl.ds(s0, 16)] = base_v

    def _sel(j, carry):
        m = cmax_v[pl.ds(0, 16)]
        for i in range(1, 8):
            m = jnp.maximum(m, cmax_v[pl.ds(i * 16, 16)])
        gmax_v = _vmax(m)
        big = jnp.full((16,), 1 << 30, jnp.int32)
        cand = big
        for i in range(8):
            cm = cmax_v[pl.ds(i * 16, 16)]
            cand = jnp.minimum(cand, jnp.where(cm >= gmax_v,
                                               i * 16 + lanes, big))
        chunk_id = _vmin(cand)[0]
        coff = pl.multiple_of(chunk_id * 16, 16)
        ch = amp_v[pl.ds(coff, 16)]
        lane = _vmin(jnp.where(ch >= gmax_v, lanes, big))[0]
        flat = chunk_id * 16 + lane
        gidx_v = jnp.full((16,), base + flat * H, jnp.int32)
        s0 = pl.multiple_of((j // 16) * 16, 16)
        islice = idx_v[pl.ds(s0, 16)]
        idx_v[pl.ds(s0, 16)] = jnp.where(lanes == j % 16, gidx_v, islice)
        newch = jnp.where(lanes == lane, neg_v, ch)
        amp_v[pl.ds(coff, 16)] = newch
        g2off = pl.multiple_of((chunk_id // 16) * 16, 16)
        cslice = cmax_v[pl.ds(g2off, 16)]
        cmax_v[pl.ds(g2off, 16)] = jnp.where(lanes == chunk_id % 16,
                                             _vmax(newch), cslice)
        return carry

    lax.fori_loop(0, N_TOP, _sel, 0)

    @pl.when(br == 0)
    def _():
        pltpu.async_copy(qv_hbm.at[idx_v], rows_v, sem).wait()
        pltpu.sync_copy(rows_v, qvsel_hbm.at[bh])

    @pl.when(br == 1)
    def _():
        pltpu.async_copy(tfv_hbm.at[idx_v], rows_v, sem).wait()
        pltpu.sync_copy(rows_v, tfvsel_hbm.at[bh])


def _topk_gather(amp_t, amp_tf, qv, tfv):
    mesh = plsc.VectorSubcoreMesh(core_axis_name="c", subcore_axis_name="s")
    sds = jax.ShapeDtypeStruct((16, N_PAD, 2 * E), jnp.float32)
    fn = pl.kernel(
        _topk_gather_body,
        mesh=mesh,
        out_type=[sds, sds],
        scratch_types=[
            pltpu.VMEM((L,), jnp.float32),
            pltpu.VMEM((_NCHUNK,), jnp.float32),
            pltpu.VMEM((N_PAD,), jnp.int32),
            pltpu.VMEM((N_PAD, 2 * E), jnp.float32),
            pltpu.SemaphoreType.DMA,
        ],
    )
    return fn(amp_t, amp_tf, qv, tfv)


def _attn_body(qv_ref, tfv_ref, k_ref, tf_ref, w_ref, out_ref):
    dnS = (((1,), (1,)), ((), ()))
    dnO = (((0,), (0,)), ((), ()))
    rows = lax.broadcasted_iota(jnp.int32, (N_PAD, L), 0)
    valid = rows < N_TOP

    def branch(sel, vsel, kmat):
        s = lax.dot_general(sel, kmat, dnS, precision=_HC) * SCALE
        s = jnp.where(valid, s, NEG)
        s = s - jnp.max(s, axis=0, keepdims=True)
        p = jnp.exp(s)
        a = p / jnp.sum(p, axis=0, keepdims=True)
        return lax.dot_general(vsel, a, dnO, precision=_HC)

    for s_ in range(2):
        kb = k_ref[0][:, s_ * E:(s_ + 1) * E]
        tfb = tf_ref[0][:, s_ * E:(s_ + 1) * E]
        qv = qv_ref[s_]
        tfv = tfv_ref[s_]
        o_t = branch(qv[:, :E], qv[:, E:], kb)
        o_tf = branch(tfv[:, :E], tfv[:, E:], tfb)
        w0 = w_ref[s_, 0, :]
        w1 = w_ref[s_, 1, :]
        nw0 = 1.0 / (1.0 + jnp.exp(w1 - w0))
        out_ref[0, s_] = o_t * nw0[None, :] + o_tf * (1.0 - nw0)[None, :]


def _attention(qv_sel, tfv_sel, xk, xtf, w_t):
    grid = (B, H // 2)
    return pl.pallas_call(
        _attn_body,
        grid=grid,
        in_specs=[
            pl.BlockSpec((2, N_PAD, 2 * E),
                         lambda b, j: (b * (H // 2) + j, 0, 0)),
            pl.BlockSpec((2, N_PAD, 2 * E),
                         lambda b, j: (b * (H // 2) + j, 0, 0)),
            pl.BlockSpec((1, L, 2 * E), lambda b, j: (b, 0, j)),
            pl.BlockSpec((1, L, 2 * E), lambda b, j: (b, 0, j)),
            pl.BlockSpec((2, 2, L), lambda b, j: (j, 0, 0)),
        ],
        out_specs=pl.BlockSpec((1, 2, E, L), lambda b, j: (b, j, 0, 0)),
        out_shape=jax.ShapeDtypeStruct((B, H, E, L), jnp.float32),
    )(qv_sel, tfv_sel, xk, xtf, w_t)


def kernel(tf_queries, queries, keys, values, mask, weights,
           weights1_real, weights1_imag):
    del mask, weights1_real, weights1_imag
    xq = queries.reshape(B, L, HE)
    xk = keys.reshape(B, L, HE)
    xtf = tf_queries.reshape(B, L, HE)
    amp_t, amp_tf = _amplitudes(xq, xk, xtf)
    v_r = values.reshape(B * L * H, E)
    qv = jnp.concatenate([queries.reshape(B * L * H, E), v_r], axis=1)
    tfv = jnp.concatenate([tf_queries.reshape(B * L * H, E), v_r], axis=1)
    qv_sel, tfv_sel = _topk_gather(
        amp_t.reshape(B * H, L), amp_tf.reshape(B * H, L), qv, tfv)
    w_t = jnp.transpose(weights, (0, 2, 1))
    return _attention(qv_sel, tfv_sel, xk, xtf, w_t)
